# baseline (speedup 1.0000x reference)
import jax
import jax.numpy as jnp
from jax import lax
from jax.experimental import pallas as pl
from jax.experimental.pallas import tpu as pltpu
from jax.experimental.pallas import tpu_sc as plsc

_NC = 2
_NS = 16
_NW = _NC * _NS
_LANES = 16


def _embed_kernel(idx_hbm, tok_hbm, pos_hbm, out_hbm, idx_v, rows_v, pos_v,
                  isem, psem, gsem, osem):
    n, embed = out_hbm.shape
    batch, seqlen = idx_hbm.shape
    seg = seqlen // _NW
    wid = lax.axis_index("s") * _NC + lax.axis_index("c")
    s0 = wid * seg

    half = seg // 2
    pos_cp = pltpu.async_copy(pos_hbm.at[pl.ds(s0, seg)], pos_v, psem)
    idx_cps = [
        pltpu.async_copy(idx_hbm.at[b, pl.ds(s0, seg)], idx_v.at[b], isem.at[b])
        for b in range(batch)
    ]
    gat = []
    for b in range(batch):
        idx_cps[b].wait()
        for h in range(2):
            gat.append(
                pltpu.async_copy(
                    tok_hbm.at[idx_v.at[b, pl.ds(h * half, half)]],
                    rows_v.at[pl.ds(b * seg + h * half, half)],
                    gsem.at[2 * b + h],
                )
            )
    pos_cp.wait()
    out = []
    for b in range(batch):
        for h in range(2):
            gat[2 * b + h].wait()

            @pl.loop(0, half)
            def _row(i, b=b, h=h):
                @pl.loop(0, embed, step=_LANES)
                def _lane(j, i=i, b=b, h=h):
                    dst = (pl.ds(b * seg + h * half + i, 1), pl.ds(j, _LANES))
                    src = (pl.ds(h * half + i, 1), pl.ds(j, _LANES))
                    rows_v.at[*dst][...] = rows_v.at[*dst][...] + pos_v.at[*src][...]

            out.append(
                pltpu.async_copy(
                    rows_v.at[pl.ds(b * seg + h * half, half)],
                    out_hbm.at[pl.ds(b * seqlen + s0 + h * half, half)],
                    osem.at[2 * b + h],
                )
            )
    for k in range(2 * batch):
        out[k].wait()


def kernel(inputs, input_table, position_table):
    batch, seqlen = inputs.shape
    vocab, embed = input_table.shape
    n = batch * seqlen
    seg = seqlen // _NW

    mesh = plsc.VectorSubcoreMesh(core_axis_name="c", subcore_axis_name="s")
    run = pl.kernel(
        _embed_kernel,
        out_type=jax.ShapeDtypeStruct((n, embed), jnp.float32),
        mesh=mesh,
        scratch_types=[
            pltpu.VMEM((batch, seg), jnp.int32),
            pltpu.VMEM((batch * seg, embed), jnp.float32),
            pltpu.VMEM((seg, embed), jnp.float32),
            pltpu.SemaphoreType.DMA((batch,)),
            pltpu.SemaphoreType.DMA,
            pltpu.SemaphoreType.DMA((2 * batch,)),
            pltpu.SemaphoreType.DMA((2 * batch,)),
        ],
    )
    out = run(inputs.astype(jnp.int32), input_table, position_table)
    return out.reshape(batch, seqlen, embed)


# speedup vs baseline: 1.0108x; 1.0108x over previous
import jax
import jax.numpy as jnp
from jax import lax
from jax.experimental import pallas as pl
from jax.experimental.pallas import tpu as pltpu
from jax.experimental.pallas import tpu_sc as plsc

_NC = 2
_NS = 16
_NW = _NC * _NS
_LANES = 16


def _embed_kernel(idx_hbm, tok_hbm, pos_hbm, out_hbm, idx_v, rows_v, pos_v,
                  isem, psem, gsem, osem):
    n, embed = out_hbm.shape
    batch, seqlen = idx_hbm.shape
    seg = seqlen // _NW
    wid = lax.axis_index("s") * _NC + lax.axis_index("c")
    s0 = wid * seg

    pos_cp = pltpu.async_copy(pos_hbm.at[pl.ds(s0, seg)], pos_v, psem)
    idx_cps = [
        pltpu.async_copy(idx_hbm.at[b, pl.ds(s0, seg)], idx_v.at[b], isem.at[b])
        for b in range(batch)
    ]
    gat = []
    for b in range(batch):
        idx_cps[b].wait()
        gat.append(
            pltpu.async_copy(
                tok_hbm.at[idx_v.at[b]],
                rows_v.at[pl.ds(b * seg, seg)],
                gsem.at[b],
            )
        )
    pos_cp.wait()
    out = []
    for b in range(batch):
        gat[b].wait()

        @pl.loop(0, seg)
        def _row(i, b=b):
            @pl.loop(0, embed, step=_LANES)
            def _lane(j, i=i, b=b):
                dst = (pl.ds(b * seg + i, 1), pl.ds(j, _LANES))
                src = (pl.ds(i, 1), pl.ds(j, _LANES))
                rows_v.at[*dst][...] = rows_v.at[*dst][...] + pos_v.at[*src][...]

        out.append(
            pltpu.async_copy(
                rows_v.at[pl.ds(b * seg, seg)],
                out_hbm.at[pl.ds(b * seqlen + s0, seg)],
                osem.at[b],
            )
        )
    for b in range(batch):
        out[b].wait()


def kernel(inputs, input_table, position_table):
    batch, seqlen = inputs.shape
    vocab, embed = input_table.shape
    n = batch * seqlen
    seg = seqlen // _NW

    mesh = plsc.VectorSubcoreMesh(core_axis_name="c", subcore_axis_name="s")
    run = pl.kernel(
        _embed_kernel,
        out_type=jax.ShapeDtypeStruct((n, embed), jnp.float32),
        mesh=mesh,
        scratch_types=[
            pltpu.VMEM((batch, seg), jnp.int32),
            pltpu.VMEM((batch * seg, embed), jnp.float32),
            pltpu.VMEM((seg, embed), jnp.float32),
            pltpu.SemaphoreType.DMA((batch,)),
            pltpu.SemaphoreType.DMA,
            pltpu.SemaphoreType.DMA((batch,)),
            pltpu.SemaphoreType.DMA((batch,)),
        ],
    )
    out = run(inputs.astype(jnp.int32), input_table, position_table)
    return out.reshape(batch, seqlen, embed)
